# baseline (speedup 1.0000x reference)
import functools

import jax
import jax.numpy as jnp
from jax import lax
from jax.experimental import pallas as pl
from jax.experimental.pallas import tpu as pltpu
from jax.experimental.pallas import tpu_sc as plsc

N = 8388608
NC = 2
NS = 16
L = 16
NW = NC * NS
PER_W = N // NW
CHUNK = 16384
NCHUNK = PER_W // CHUNK
VECS = CHUNK // L

W_LO = 2.0 / 3.0


@functools.partial(
    pl.kernel,
    out_type=jax.ShapeDtypeStruct((NW, 2, L), jnp.float32),
    mesh=plsc.VectorSubcoreMesh(core_axis_name="c", subcore_axis_name="s",
                                num_cores=NC, num_subcores=NS),
    scratch_types=[
        pltpu.VMEM((CHUNK,), jnp.float32),
        pltpu.VMEM((CHUNK,), jnp.float32),
        pltpu.VMEM((CHUNK,), jnp.float32),
        pltpu.VMEM((CHUNK,), jnp.float32),
        pltpu.VMEM((2, L), jnp.float32),
        pltpu.SemaphoreType.DMA,
        pltpu.SemaphoreType.DMA,
    ],
)
def _sc_partials(pred_hbm, gt_hbm, out_hbm, p0, p1, g0, g1, stage, sem0, sem1):
    cid = lax.axis_index("c")
    sid = lax.axis_index("s")
    wid = sid * NC + cid
    base = wid * PER_W

    pbuf = (p0, p1)
    gbuf = (g0, g1)
    sems = (sem0, sem1)

    def start(k):
        slot = k % 2
        off = base + k * CHUNK
        cp = pltpu.async_copy(pred_hbm.at[pl.ds(off, CHUNK)], pbuf[slot], sems[slot])
        cg = pltpu.async_copy(gt_hbm.at[pl.ds(off, CHUNK)], gbuf[slot], sems[slot])
        return cp, cg

    inflight = {0: start(0)}
    U = 8
    zero = jnp.zeros((L,), jnp.float32)
    acc = (zero,) * (2 * U)
    for k in range(NCHUNK):
        if k + 1 < NCHUNK:
            inflight[k + 1] = start(k + 1)
        cp, cg = inflight.pop(k)
        cp.wait()
        cg.wait()
        pv = pbuf[k % 2]
        gv = gbuf[k % 2]

        @pl.loop(0, VECS // U, init_carry=acc)
        def acc_loop(j, carry):
            aws = list(carry[:U])
            awds = list(carry[U:])
            base_i = j * (U * L)
            for u in range(U):
                p = pv[pl.ds(base_i + u * L, L)]
                g = gv[pl.ds(base_i + u * L, L)]
                d = g - p
                w = jnp.where(g <= 0.55, W_LO, 1.0)
                aws[u] = aws[u] + w
                awds[u] = awds[u] + w * (d * d)
            return tuple(aws) + tuple(awds)

        acc = acc_loop

    aw_t = ((acc[0] + acc[1]) + (acc[2] + acc[3])) + ((acc[4] + acc[5]) + (acc[6] + acc[7]))
    awd_t = ((acc[8] + acc[9]) + (acc[10] + acc[11])) + ((acc[12] + acc[13]) + (acc[14] + acc[15]))
    stage[0, :] = aw_t
    stage[1, :] = awd_t
    pltpu.sync_copy(stage, out_hbm.at[wid])


def kernel(y_pred, y_gt):
    partials = _sc_partials(y_pred, y_gt)
    sums = jnp.sum(partials, axis=(0, 2))
    return sums[1] / sums[0]


# speedup vs baseline: 1.0266x; 1.0266x over previous
import functools

import jax
import jax.numpy as jnp
from jax import lax
from jax.experimental import pallas as pl
from jax.experimental.pallas import tpu as pltpu
from jax.experimental.pallas import tpu_sc as plsc

N = 8388608
NC = 2
NS = 16
L = 16
NW = NC * NS
PER_W = N // NW
CHUNK = 16384
NCHUNK = PER_W // CHUNK
VECS = CHUNK // L

W_LO = 2.0 / 3.0


@functools.partial(
    pl.kernel,
    out_type=jax.ShapeDtypeStruct((NW, 2, L), jnp.float32),
    mesh=plsc.VectorSubcoreMesh(core_axis_name="c", subcore_axis_name="s",
                                num_cores=NC, num_subcores=NS),
    scratch_types=[
        pltpu.VMEM((CHUNK,), jnp.float32),
        pltpu.VMEM((CHUNK,), jnp.float32),
        pltpu.VMEM((CHUNK,), jnp.float32),
        pltpu.VMEM((CHUNK,), jnp.float32),
        pltpu.VMEM((2, L), jnp.float32),
        pltpu.SemaphoreType.DMA,
        pltpu.SemaphoreType.DMA,
    ],
)
def _sc_partials(pred_hbm, gt_hbm, out_hbm, p0, p1, g0, g1, stage, sem0, sem1):
    cid = lax.axis_index("c")
    sid = lax.axis_index("s")
    wid = sid * NC + cid
    base = wid * PER_W

    pbuf = (p0, p1)
    gbuf = (g0, g1)
    sems = (sem0, sem1)

    def start(slot, off):
        pltpu.async_copy(pred_hbm.at[pl.ds(off, CHUNK)], pbuf[slot], sems[slot])
        pltpu.async_copy(gt_hbm.at[pl.ds(off, CHUNK)], gbuf[slot], sems[slot])

    def wait(slot):
        pltpu.make_async_copy(pred_hbm.at[pl.ds(base, CHUNK)], pbuf[slot], sems[slot]).wait()
        pltpu.make_async_copy(gt_hbm.at[pl.ds(base, CHUNK)], gbuf[slot], sems[slot]).wait()

    U = 8

    def compute(pv, gv, acc):
        @pl.loop(0, VECS // U, init_carry=acc)
        def acc_loop(j, carry):
            aws = list(carry[:U])
            awds = list(carry[U:])
            base_i = j * (U * L)
            for u in range(U):
                p = pv[pl.ds(base_i + u * L, L)]
                g = gv[pl.ds(base_i + u * L, L)]
                d = g - p
                w = jnp.where(g <= 0.55, W_LO, 1.0)
                aws[u] = aws[u] + w
                awds[u] = awds[u] + w * (d * d)
            return tuple(aws) + tuple(awds)

        return acc_loop

    zero = jnp.zeros((L,), jnp.float32)
    acc = (zero,) * (2 * U)
    start(0, base)
    start(1, base + CHUNK)

    @pl.loop(0, NCHUNK // 2 - 1, init_carry=acc)
    def chunk_loop(kk, acc):
        for b in range(2):
            k = 2 * kk + b
            wait(b)
            acc = compute(pbuf[b], gbuf[b], acc)
            start(b, base + (k + 2) * CHUNK)
        return acc

    acc = chunk_loop
    for b in range(2):
        wait(b)
        acc = compute(pbuf[b], gbuf[b], acc)

    aw_t = ((acc[0] + acc[1]) + (acc[2] + acc[3])) + ((acc[4] + acc[5]) + (acc[6] + acc[7]))
    awd_t = ((acc[8] + acc[9]) + (acc[10] + acc[11])) + ((acc[12] + acc[13]) + (acc[14] + acc[15]))
    stage[0, :] = aw_t
    stage[1, :] = awd_t
    pltpu.sync_copy(stage, out_hbm.at[wid])


def kernel(y_pred, y_gt):
    partials = _sc_partials(y_pred, y_gt)
    sums = jnp.sum(partials, axis=(0, 2))
    return sums[1] / sums[0]
